# baseline (speedup 1.0000x reference)
import functools
import jax
import jax.numpy as jnp
from jax import lax
from jax.experimental import pallas as pl
from jax.experimental.pallas import tpu as pltpu
from jax.experimental.pallas import tpu_sc as plsc

_NW = 32
_R = 4096
_C = 1024
_RPW = _R // _NW
_RB = 4
_NB = _RPW // _RB
_BR = 512


@functools.partial(
    pl.kernel,
    mesh=plsc.VectorSubcoreMesh(core_axis_name="c", subcore_axis_name="s"),
    out_type=jax.ShapeDtypeStruct((3, _R, _C), jnp.float32),
    scratch_types=[
        pltpu.VMEM((1, _RB, _C), jnp.float32),
        pltpu.SemaphoreType.DMA,
    ],
)
def _sc_memset(out_hbm, cbuf, sem):
    wid = lax.axis_index("s") * 2 + lax.axis_index("c")
    for r in range(_RB):
        for c in range(_C // 16):
            cbuf[0, r, pl.ds(c * 16, 16)] = jnp.full((16,), 2.0, jnp.float32)
    row0 = wid * _RPW
    hs = []
    for b in range(_NB):
        dst = out_hbm.at[pl.ds(0, 1), pl.ds(row0 + b * _RB, _RB), :]
        hs.append(pltpu.async_copy(cbuf, dst, sem))
    for h in hs:
        h.wait()


def _tc_body(x_ref, scaf_ref, o1_ref, o2_ref):
    del scaf_ref
    s = pl.program_id(1)
    v = x_ref[...]
    o1_ref[...] = v

    @pl.when(s >= 1)
    def _():
        o2_ref[...] = v


def kernel(x):
    scaffold = _sc_memset()
    blk = (1, _BR, _C)
    out1, out2 = pl.pallas_call(
        _tc_body,
        grid=(_R // _BR, 3),
        in_specs=[
            pl.BlockSpec(blk, lambda i, s: (s, i, 0)),
            pl.BlockSpec(memory_space=pl.ANY),
        ],
        out_specs=[
            pl.BlockSpec(blk, lambda i, s: (s, i, 0)),
            pl.BlockSpec(blk, lambda i, s: (jnp.maximum(s, 1), i, 0)),
        ],
        out_shape=[
            jax.ShapeDtypeStruct(x.shape, x.dtype),
            jax.ShapeDtypeStruct(x.shape, x.dtype),
        ],
        input_output_aliases={1: 1},
    )(x, scaffold)
    return (out1, out2)


# speedup vs baseline: 1.5988x; 1.5988x over previous
import jax
import jax.numpy as jnp
from jax.experimental import pallas as pl


_BR = 2048
_SLICE_ROWS = 4096


def _body(x_ref, o1_ref, o2_ref):
    i = pl.program_id(0)
    v = x_ref[...]
    o1_ref[...] = v
    row = i * _BR + jax.lax.broadcasted_iota(jnp.int32, v.shape, 0)
    o2_ref[...] = jnp.where(row < _SLICE_ROWS, jnp.float32(2.0), v)


def kernel(x):
    n, r, c = x.shape
    xf = x.reshape(n * r, c)
    grid = ((n * r) // _BR,)
    spec = pl.BlockSpec((_BR, c), lambda i: (i, 0))
    out1, out2 = pl.pallas_call(
        _body,
        grid=grid,
        in_specs=[spec],
        out_specs=[spec, spec],
        out_shape=[
            jax.ShapeDtypeStruct(xf.shape, x.dtype),
            jax.ShapeDtypeStruct(xf.shape, x.dtype),
        ],
    )(xf)
    return (out1.reshape(x.shape), out2.reshape(x.shape))
